# baseline (speedup 1.0000x reference)
import jax
import jax.numpy as jnp
import numpy as np
from jax.experimental import pallas as pl

B, N, K, NB, C = 8, 1024, 12, 14, 64
EPS = 1e-5


def _pairwise_neg_dist(x):
    x_inner = -2.0 * jnp.matmul(x, jnp.swapaxes(x, 1, 2))
    x_sq = jnp.sum(x * x, axis=-1, keepdims=True)
    return -(x_sq + x_inner + jnp.swapaxes(x_sq, 1, 2))


def _knn(x, k):
    _, idx = jax.lax.top_k(_pairwise_neg_dist(x), k)
    return idx


def _gather(x, idx):
    return jax.vmap(lambda xb, ib: xb[ib])(x, idx)


def _bn(y, gamma, beta, axes):
    mean = jnp.mean(y, axis=axes, keepdims=True)
    var = jnp.var(y, axis=axes, keepdims=True)
    return (y - mean) / jnp.sqrt(var + EPS) * gamma + beta


def _edge_conv(x, idx, W, b, gamma, beta):
    x_j = _gather(x, idx)
    x_i = jnp.broadcast_to(x[:, :, None, :], x_j.shape)
    feat = jnp.concatenate([x_i, x_j - x_i], axis=-1)
    y = jnp.einsum('bnkc,oc->bnko', feat, W)
    if b is not None:
        y = y + b
    y = _bn(y, gamma, beta, (0, 1, 2))
    y = jax.nn.relu(y)
    return jnp.max(y, axis=2)


def _fusion_mm_kernel(f_ref, w_ref, o_ref):
    o_ref[...] = jnp.dot(f_ref[...], w_ref[...],
                         preferred_element_type=jnp.float32)


def _fusion_matmul(fusion_in, W_fus):
    M = fusion_in.shape[0]
    return pl.pallas_call(
        _fusion_mm_kernel,
        grid=(M // 512,),
        in_specs=[pl.BlockSpec((512, 896), lambda i: (i, 0)),
                  pl.BlockSpec((896, 1024), lambda i: (0, 0))],
        out_specs=pl.BlockSpec((512, 1024), lambda i: (i, 0)),
        out_shape=jax.ShapeDtypeStruct((M, 1024), jnp.float32),
    )(fusion_in, W_fus.T)


def kernel(points, features, W_head, g_head, beta_head, W_blocks, b_blocks, g_blocks, beta_blocks, W_fus, g_fus, beta_fus, W_p1, b_p1, g_p1, beta_p1, W_p2, b_p2, g_p2, beta_p2, W_p3, b_p3, W_proj, b_proj):
    pts = jnp.swapaxes(points, 1, 2)
    x = jnp.swapaxes(features, 1, 2)
    idx = _knn(pts, K)
    feats = [_edge_conv(x, idx, W_head, None, g_head, beta_head)]
    for i in range(NB - 1):
        d = i + 1
        h = feats[-1]
        idx_full = _knn(h, K * d)
        idx_d = idx_full[:, :, ::d]
        feats.append(_edge_conv(h, idx_d, W_blocks[i], b_blocks[i], g_blocks[i], beta_blocks[i]) + h)
    fusion_in = jnp.concatenate(feats, axis=-1)
    y = _fusion_matmul(fusion_in.reshape(B * N, -1), W_fus).reshape(B, N, 1024)
    y = _bn(y, g_fus, beta_fus, (0, 1))
    y = jax.nn.leaky_relu(y, 0.2)
    x1 = jnp.max(y, axis=1)
    x2 = jnp.mean(y, axis=1)
    h = jnp.concatenate([x1, x2], axis=-1)
    h = jax.nn.leaky_relu(_bn(h @ W_p1.T + b_p1, g_p1, beta_p1, (0,)), 0.2)
    h = jax.nn.leaky_relu(_bn(h @ W_p2.T + b_p2, g_p2, beta_p2, (0,)), 0.2)
    out = h @ W_p3.T + b_p3
    proj = out @ W_proj.T + b_proj
    return (proj, out)
